# baseline (speedup 1.0000x reference)
import functools
import math

import jax
import jax.numpy as jnp
from jax import lax
from jax.experimental import pallas as pl
from jax.experimental.pallas import tpu as pltpu
from jax.experimental.pallas import tpu_sc as plsc

D_MODEL = 64
SCALE = math.sqrt(D_MODEL)

NC = 2
NS = 16
NW = NC * NS

SB = 128
NBUF = 4
LANES = 16


def _emb_body(x4_hbm, table_hbm, out_hbm, idx_v, bufs, gsems, ssems):
    wid = lax.axis_index("s") * NC + lax.axis_index("c")
    ntok = x4_hbm.shape[0] * x4_hbm.shape[2]
    seq0 = wid * SB

    pltpu.sync_copy(x4_hbm.at[:, wid], idx_v)

    def start_gather(b, t):
        rb = t // 8
        rr = t % 8
        pltpu.async_copy(table_hbm.at[idx_v.at[rb, rr]], bufs[b], gsems[b])

    def start_store(b, t):
        pltpu.async_copy(bufs[b], out_hbm.at[pl.ds(seq0, SB), t], ssems[b])

    def wait_store(b, t):
        pltpu.make_async_copy(bufs[b], out_hbm.at[pl.ds(seq0, SB), t],
                              ssems[b]).wait()

    for b in range(NBUF - 1):
        start_gather(b, b)

    def round_body(r):
        for b in range(NBUF):
            t = r * NBUF + b

            rb = t // 8
            rr = t % 8
            pltpu.make_async_copy(table_hbm.at[idx_v.at[rb, rr]], bufs[b],
                                  gsems[b]).wait()

            def scale_row(row, _):
                for c in range(D_MODEL // LANES):
                    sl = pl.ds(c * LANES, LANES)
                    bufs[b][row, sl] = bufs[b][row, sl] * SCALE
                return 0

            lax.fori_loop(0, SB, scale_row, 0, unroll=4)

            start_store(b, t)

            bp = (b - 1) % NBUF
            tp = t - 1

            @pl.when(tp >= 0)
            def _():
                wait_store(bp, tp)

            @pl.when(tp + NBUF < ntok)
            def _():
                start_gather(bp, tp + NBUF)

    pl.loop(0, ntok // NBUF)(round_body)

    wait_store((ntok - 1) % NBUF, ntok - 1)


@jax.jit
def _emb_call(x4, table):
    ntok = x4.shape[0] * x4.shape[2]
    nseq = x4.shape[1] * x4.shape[3]
    mesh = plsc.VectorSubcoreMesh(core_axis_name="c", subcore_axis_name="s",
                                  num_cores=NC, num_subcores=NS)
    scratch = (
        [pltpu.VMEM((x4.shape[0], x4.shape[2], SB), jnp.int32)]
        + [[pltpu.VMEM((SB, D_MODEL), jnp.float32) for _ in range(NBUF)]]
        + [[pltpu.SemaphoreType.DMA for _ in range(NBUF)]]
        + [[pltpu.SemaphoreType.DMA for _ in range(NBUF)]]
    )
    kern = pl.kernel(
        _emb_body,
        out_type=jax.ShapeDtypeStruct((nseq, ntok, D_MODEL), jnp.float32),
        mesh=mesh,
        scratch_types=scratch,
        compiler_params=pltpu.CompilerParams(use_tc_tiling_on_sc=False),
    )
    return kern(x4, table)


def kernel(x, table):
    nseq, ntok = x.shape
    x4 = x.T.reshape(ntok // 8, 8, nseq // SB, SB).transpose(0, 2, 1, 3)
    return _emb_call(x4, table)
